# Initial kernel scaffold; baseline (speedup 1.0000x reference)
import jax
import jax.numpy as jnp
from jax.experimental import pallas as pl


def kernel(x, spirals0, W_enc, b_enc, D0, W_fc_enc, b_fc_enc, W_fc_dec, b_fc_dec, U0, W_dec, b_dec):
    raise NotImplementedError("write your pallas kernel here")


# speedup vs baseline: 9.6594x; 9.6594x over previous
import functools

import jax
import jax.numpy as jnp
from jax import lax
from jax.experimental import pallas as pl
from jax.experimental.pallas import tpu as pltpu, tpu_sc as plsc

B = 8
P0 = 16384
P1 = 1024
S = 16
F_IN = 16
F_ENC = 128
LATENT = 256
F_DEC0 = 128
F_OUT = 16

NW = 32
_ENC_ROWS = B * P0 * S
_DEC_ROWS = B * P0
_STREAM = 128


def _sc_mesh():
    return plsc.VectorSubcoreMesh(core_axis_name="c", subcore_axis_name="s")


def _worker_id():
    return lax.axis_index("s") * 2 + lax.axis_index("c")


_ENC_PER_W = _ENC_ROWS // NW
_ENC_CHUNK = 2048
_ENC_ITERS = _ENC_PER_W // _ENC_CHUNK
_ENC_FIRES = _ENC_CHUNK // _STREAM


@functools.partial(
    pl.kernel,
    mesh=_sc_mesh(),
    out_type=jax.ShapeDtypeStruct((_ENC_ROWS, F_IN), jnp.float32),
    compiler_params=pltpu.CompilerParams(use_tc_tiling_on_sc=False),
    scratch_types=[
        pltpu.VMEM((_ENC_CHUNK,), jnp.int32),
        pltpu.VMEM((_ENC_CHUNK, F_IN), jnp.float32),
        pltpu.SemaphoreType.DMA,
    ],
)
def _sc_enc_gather(table_hbm, idx_hbm, out_hbm, idx_v, rows_v, sem):
    wid = _worker_id()

    def body(i, carry):
        base = wid * _ENC_PER_W + i * _ENC_CHUNK
        pltpu.sync_copy(idx_hbm.at[pl.ds(base, _ENC_CHUNK)], idx_v)
        copies = []
        for j in range(_ENC_FIRES):
            copies.append(pltpu.async_copy(
                table_hbm.at[idx_v.at[pl.ds(j * _STREAM, _STREAM)]],
                rows_v.at[pl.ds(j * _STREAM, _STREAM)],
                sem))
        for c in copies:
            c.wait()
        pltpu.sync_copy(rows_v, out_hbm.at[pl.ds(base, _ENC_CHUNK)])
        return carry

    lax.fori_loop(0, _ENC_ITERS, body, 0)


_DEC_PER_W = _DEC_ROWS // NW
_DEC_CHUNK = 128
_DEC_ITERS = _DEC_PER_W // _DEC_CHUNK
_DEC_GROWS = _DEC_CHUNK * S
_DEC_FIRES = _DEC_GROWS // _STREAM


@functools.partial(
    pl.kernel,
    mesh=_sc_mesh(),
    out_type=jax.ShapeDtypeStruct((_DEC_ROWS, F_OUT), jnp.float32),
    compiler_params=pltpu.CompilerParams(use_tc_tiling_on_sc=False),
    scratch_types=[
        pltpu.VMEM((_DEC_GROWS,), jnp.int32),
        pltpu.VMEM((_DEC_GROWS, F_OUT), jnp.float32),
        pltpu.VMEM((_DEC_CHUNK, F_OUT), jnp.float32),
        pltpu.VMEM((F_OUT,), jnp.float32),
        pltpu.SemaphoreType.DMA,
    ],
)
def _sc_dec_bag(table_hbm, idx_hbm, bdec_hbm, out_hbm,
                idx_v, rows_v, out_v, bias_v, sem):
    wid = _worker_id()
    pltpu.sync_copy(bdec_hbm, bias_v)

    def body(i, carry):
        base_out = wid * _DEC_PER_W + i * _DEC_CHUNK
        base_idx = base_out * S
        pltpu.sync_copy(idx_hbm.at[pl.ds(base_idx, _DEC_GROWS)], idx_v)
        copies = []
        for j in range(_DEC_FIRES):
            copies.append(pltpu.async_copy(
                table_hbm.at[idx_v.at[pl.ds(j * _STREAM, _STREAM)]],
                rows_v.at[pl.ds(j * _STREAM, _STREAM)],
                sem))
        for c in copies:
            c.wait()
        bias = bias_v[...]

        def row_body(r, rcarry):
            acc = bias
            for s in range(S):
                acc = acc + rows_v[r * S + s]
            keep = jnp.where((base_out + r) % P0 == P0 - 1,
                             jnp.float32(0.0), jnp.float32(1.0))
            out_v[r] = acc * keep
            return rcarry

        lax.fori_loop(0, _DEC_CHUNK, row_body, 0)
        pltpu.sync_copy(out_v, out_hbm.at[pl.ds(base_out, _DEC_CHUNK)])
        return carry

    lax.fori_loop(0, _DEC_ITERS, body, 0)


_TKA = 2048
_KBA = P0 // _TKA


def _tc_a_body(g_ref, w_ref, b_ref, d_ref, o_ref):
    k = pl.program_id(0)
    b = pl.program_id(1)
    g = g_ref[0]
    h = lax.dot_general(g, w_ref[...], (((1,), (1,)), ((), ())),
                        preferred_element_type=jnp.float32)
    h = h + b_ref[...]
    h = jnp.where(h > 0, h, jnp.exp(jnp.minimum(h, 0.0)) - 1.0)
    c = lax.dot_general(d_ref[...], h, (((1,), (0,)), ((), ())),
                        preferred_element_type=jnp.float32)

    @pl.when(k == 0)
    def _():
        o_ref[b] = c

    @pl.when(k != 0)
    def _():
        o_ref[b] = o_ref[b] + c


def _tc_a(g0r, w_enc, b_enc2, d0m):
    return pl.pallas_call(
        _tc_a_body,
        grid=(_KBA, B),
        in_specs=[
            pl.BlockSpec((1, _TKA, S * F_IN), lambda k, b: (b, k, 0)),
            pl.BlockSpec((F_ENC, S * F_IN), lambda k, b: (0, 0)),
            pl.BlockSpec((1, F_ENC), lambda k, b: (0, 0)),
            pl.BlockSpec((P1, _TKA), lambda k, b: (0, k)),
        ],
        out_specs=pl.BlockSpec((B, P1, F_ENC), lambda k, b: (0, 0, 0)),
        out_shape=jax.ShapeDtypeStruct((B, P1, F_ENC), jnp.float32),
    )(g0r, w_enc, b_enc2, d0m)


_CKB = 8192
_KBB = (P1 * F_ENC) // _CKB


def _tc_b1_body(h_ref, w_ref, b_ref, o_ref):
    k = pl.program_id(0)
    c = lax.dot_general(h_ref[...], w_ref[...], (((1,), (1,)), ((), ())),
                        preferred_element_type=jnp.float32)

    @pl.when(k == 0)
    def _():
        o_ref[...] = c + b_ref[...]

    @pl.when(k != 0)
    def _():
        o_ref[...] = o_ref[...] + c


def _tc_b1(hd_flat, w_fc_enc, b_fc_enc2):
    return pl.pallas_call(
        _tc_b1_body,
        grid=(_KBB,),
        in_specs=[
            pl.BlockSpec((B, _CKB), lambda k: (0, k)),
            pl.BlockSpec((LATENT, _CKB), lambda k: (0, k)),
            pl.BlockSpec((1, LATENT), lambda k: (0, 0)),
        ],
        out_specs=pl.BlockSpec((B, LATENT), lambda k: (0, 0)),
        out_shape=jax.ShapeDtypeStruct((B, LATENT), jnp.float32),
    )(hd_flat, w_fc_enc, b_fc_enc2)


_CNB = 8192
_NBB = (P1 * F_DEC0) // _CNB


def _tc_b2_body(z_ref, w_ref, b_ref, o_ref):
    c = lax.dot_general(z_ref[...], w_ref[...], (((1,), (1,)), ((), ())),
                        preferred_element_type=jnp.float32)
    o_ref[...] = c + b_ref[...]


def _tc_b2(z, w_fc_dec, b_fc_dec2):
    return pl.pallas_call(
        _tc_b2_body,
        grid=(_NBB,),
        in_specs=[
            pl.BlockSpec((B, LATENT), lambda n: (0, 0)),
            pl.BlockSpec((_CNB, LATENT), lambda n: (n, 0)),
            pl.BlockSpec((1, _CNB), lambda n: (0, n)),
        ],
        out_specs=pl.BlockSpec((B, _CNB), lambda n: (0, n)),
        out_shape=jax.ShapeDtypeStruct((B, P1 * F_DEC0), jnp.float32),
    )(z, w_fc_dec, b_fc_dec2)


_TMC = 2048
_MBC = P0 // _TMC


def _tc_c_body(u_ref, d_ref, w_ref, o_ref):
    u = lax.dot_general(u_ref[...], d_ref[0], (((1,), (0,)), ((), ())),
                        preferred_element_type=jnp.float32)
    y = lax.dot_general(u, w_ref[...], (((1,), (0,)), ((), ())),
                        preferred_element_type=jnp.float32)
    o_ref[0] = y


def _tc_c(u0, dd3, wd2):
    return pl.pallas_call(
        _tc_c_body,
        grid=(_MBC, B),
        in_specs=[
            pl.BlockSpec((_TMC, P1), lambda m, b: (m, 0)),
            pl.BlockSpec((1, P1, F_DEC0), lambda m, b: (b, 0, 0)),
            pl.BlockSpec((F_DEC0, S * F_OUT), lambda m, b: (0, 0)),
        ],
        out_specs=pl.BlockSpec((1, _TMC, S * F_OUT), lambda m, b: (b, m, 0)),
        out_shape=jax.ShapeDtypeStruct((B, P0, S * F_OUT), jnp.float32),
    )(u0, dd3, wd2)


def kernel(x, spirals0, W_enc, b_enc, D0, W_fc_enc, b_fc_enc, W_fc_dec,
           b_fc_dec, U0, W_dec, b_dec):
    sp = spirals0.astype(jnp.int32)
    boff = (jnp.arange(B, dtype=jnp.int32) * P0)
    enc_idx = (sp.reshape(-1)[None, :] + boff[:, None]).reshape(-1)
    dec_idx = ((sp[None, :, :] + boff[:, None, None]) * S
               + jnp.arange(S, dtype=jnp.int32)[None, None, :]).reshape(-1)

    d0m = D0.at[:, P0 - 1].set(0.0)
    wd2 = W_dec.reshape(F_OUT, S, F_DEC0).transpose(2, 1, 0).reshape(
        F_DEC0, S * F_OUT)

    g0 = _sc_enc_gather(x.reshape(B * P0, F_IN), enc_idx)
    hd = _tc_a(g0.reshape(B, P0, S * F_IN), W_enc, b_enc.reshape(1, F_ENC),
               d0m)
    z = _tc_b1(hd.reshape(B, P1 * F_ENC), W_fc_enc,
               b_fc_enc.reshape(1, LATENT))
    dd = _tc_b2(z, W_fc_dec, b_fc_dec.reshape(1, P1 * F_DEC0))
    y = _tc_c(U0, dd.reshape(B, P1, F_DEC0), wd2)
    out = _sc_dec_bag(y.reshape(B * P0 * S, F_OUT), dec_idx, b_dec)
    return out.reshape(B, P0, F_OUT)


# speedup vs baseline: 10.3063x; 1.0670x over previous
import functools

import jax
import jax.numpy as jnp
from jax import lax
from jax.experimental import pallas as pl
from jax.experimental.pallas import tpu as pltpu, tpu_sc as plsc

B = 8
P0 = 16384
P1 = 1024
S = 16
F_IN = 16
F_ENC = 128
LATENT = 256
F_DEC0 = 128
F_OUT = 16

NW = 32
_STREAM = 128
_PW = P0 // NW
_PCH = 128
_NCH = _PW // _PCH
_GROWS = _PCH * S
_FIRES = _GROWS // _STREAM


def _sc_mesh():
    return plsc.VectorSubcoreMesh(core_axis_name="c", subcore_axis_name="s")


def _worker_id():
    return lax.axis_index("s") * 2 + lax.axis_index("c")


@functools.partial(
    pl.kernel,
    mesh=_sc_mesh(),
    out_type=jax.ShapeDtypeStruct((B * P0 * S, F_IN), jnp.float32),
    compiler_params=pltpu.CompilerParams(use_tc_tiling_on_sc=False),
    scratch_types=[
        pltpu.VMEM((_GROWS,), jnp.int32),
        pltpu.VMEM((_GROWS, F_IN), jnp.float32),
        pltpu.SemaphoreType.DMA,
    ],
)
def _sc_enc_gather(x_hbm, sp_hbm, out_hbm, sp_v, rows_v, sem):
    wid = _worker_id()

    def chunk_body(pc, carry):
        pbase = wid * _PW + pc * _PCH
        pltpu.sync_copy(sp_hbm.at[pl.ds(pbase * S, _GROWS)], sp_v)
        for b in range(B):
            copies = []
            for j in range(_FIRES):
                copies.append(pltpu.async_copy(
                    x_hbm.at[b].at[sp_v.at[pl.ds(j * _STREAM, _STREAM)]],
                    rows_v.at[pl.ds(j * _STREAM, _STREAM)],
                    sem))
            for c in copies:
                c.wait()
            pltpu.sync_copy(
                rows_v, out_hbm.at[pl.ds((b * P0 + pbase) * S, _GROWS)])
        return carry

    lax.fori_loop(0, _NCH, chunk_body, 0)


@functools.partial(
    pl.kernel,
    mesh=_sc_mesh(),
    out_type=jax.ShapeDtypeStruct((B, P0, F_OUT), jnp.float32),
    compiler_params=pltpu.CompilerParams(use_tc_tiling_on_sc=False),
    scratch_types=[
        pltpu.VMEM((_GROWS,), jnp.int32),
        pltpu.VMEM((_GROWS, F_OUT), jnp.float32),
        pltpu.VMEM((_PCH, F_OUT), jnp.float32),
        pltpu.VMEM((F_OUT,), jnp.float32),
        pltpu.SemaphoreType.DMA,
    ],
)
def _sc_dec_bag(y_hbm, sp_hbm, bdec_hbm, out_hbm,
                idx_v, rows_v, out_v, bias_v, sem):
    wid = _worker_id()
    pltpu.sync_copy(bdec_hbm, bias_v)
    iot = lax.iota(jnp.int32, 16)

    def chunk_body(pc, carry):
        pbase = wid * _PW + pc * _PCH
        pltpu.sync_copy(sp_hbm.at[pl.ds(pbase * S, _GROWS)], idx_v)

        def mk_idx(r, rcarry):
            sl = pl.ds(r * S, S)
            idx_v[sl] = idx_v[sl] * S + iot
            return rcarry

        lax.fori_loop(0, _PCH, mk_idx, 0)
        bias = bias_v[...]
        for b in range(B):
            if b > 0:
                def bump(r, rcarry):
                    sl = pl.ds(r * S, S)
                    idx_v[sl] = idx_v[sl] + (P0 * S)
                    return rcarry
                lax.fori_loop(0, _PCH, bump, 0)
            copies = []
            for j in range(_FIRES):
                copies.append(pltpu.async_copy(
                    y_hbm.at[idx_v.at[pl.ds(j * _STREAM, _STREAM)]],
                    rows_v.at[pl.ds(j * _STREAM, _STREAM)],
                    sem))
            for c in copies:
                c.wait()

            def row_body(r, rcarry):
                acc = bias
                for s in range(S):
                    acc = acc + rows_v[r * S + s]
                keep = jnp.where(pbase + r == P0 - 1,
                                 jnp.float32(0.0), jnp.float32(1.0))
                out_v[r] = acc * keep
                return rcarry

            lax.fori_loop(0, _PCH, row_body, 0)
            pltpu.sync_copy(out_v, out_hbm.at[b].at[pl.ds(pbase, _PCH)])
        return carry

    lax.fori_loop(0, _NCH, chunk_body, 0)


_TKA = 2048
_KBA = P0 // _TKA


def _tc_a_body(g_ref, w_ref, b_ref, d_ref, o_ref):
    k = pl.program_id(0)
    b = pl.program_id(1)
    g = g_ref[0]
    h = lax.dot_general(g, w_ref[...], (((1,), (1,)), ((), ())),
                        preferred_element_type=jnp.float32)
    h = h + b_ref[...]
    h = jnp.where(h > 0, h, jnp.exp(jnp.minimum(h, 0.0)) - 1.0)
    c = lax.dot_general(d_ref[...], h, (((1,), (0,)), ((), ())),
                        preferred_element_type=jnp.float32)

    @pl.when(k == 0)
    def _():
        o_ref[b] = c

    @pl.when(k != 0)
    def _():
        o_ref[b] = o_ref[b] + c


def _tc_a(g0r, w_enc, b_enc2, d0m):
    return pl.pallas_call(
        _tc_a_body,
        grid=(_KBA, B),
        in_specs=[
            pl.BlockSpec((1, _TKA, S * F_IN), lambda k, b: (b, k, 0)),
            pl.BlockSpec((F_ENC, S * F_IN), lambda k, b: (0, 0)),
            pl.BlockSpec((1, F_ENC), lambda k, b: (0, 0)),
            pl.BlockSpec((P1, _TKA), lambda k, b: (0, k)),
        ],
        out_specs=pl.BlockSpec((B, P1, F_ENC), lambda k, b: (0, 0, 0)),
        out_shape=jax.ShapeDtypeStruct((B, P1, F_ENC), jnp.float32),
    )(g0r, w_enc, b_enc2, d0m)


_CKB = 8192
_KBB = (P1 * F_ENC) // _CKB


def _tc_b1_body(h_ref, w_ref, b_ref, o_ref):
    k = pl.program_id(0)
    c = lax.dot_general(h_ref[...], w_ref[...], (((1,), (1,)), ((), ())),
                        preferred_element_type=jnp.float32)

    @pl.when(k == 0)
    def _():
        o_ref[...] = c + b_ref[...]

    @pl.when(k != 0)
    def _():
        o_ref[...] = o_ref[...] + c


def _tc_b1(hd_flat, w_fc_enc, b_fc_enc2):
    return pl.pallas_call(
        _tc_b1_body,
        grid=(_KBB,),
        in_specs=[
            pl.BlockSpec((B, _CKB), lambda k: (0, k)),
            pl.BlockSpec((LATENT, _CKB), lambda k: (0, k)),
            pl.BlockSpec((1, LATENT), lambda k: (0, 0)),
        ],
        out_specs=pl.BlockSpec((B, LATENT), lambda k: (0, 0)),
        out_shape=jax.ShapeDtypeStruct((B, LATENT), jnp.float32),
    )(hd_flat, w_fc_enc, b_fc_enc2)


_CNB = 8192
_NBB = (P1 * F_DEC0) // _CNB


def _tc_b2_body(z_ref, w_ref, b_ref, o_ref):
    c = lax.dot_general(z_ref[...], w_ref[...], (((1,), (1,)), ((), ())),
                        preferred_element_type=jnp.float32)
    o_ref[...] = c + b_ref[...]


def _tc_b2(z, w_fc_dec, b_fc_dec2):
    return pl.pallas_call(
        _tc_b2_body,
        grid=(_NBB,),
        in_specs=[
            pl.BlockSpec((B, LATENT), lambda n: (0, 0)),
            pl.BlockSpec((_CNB, LATENT), lambda n: (n, 0)),
            pl.BlockSpec((1, _CNB), lambda n: (0, n)),
        ],
        out_specs=pl.BlockSpec((B, _CNB), lambda n: (0, n)),
        out_shape=jax.ShapeDtypeStruct((B, P1 * F_DEC0), jnp.float32),
    )(z, w_fc_dec, b_fc_dec2)


_TMC = 2048
_MBC = P0 // _TMC


def _tc_c_body(u_ref, d_ref, w_ref, o_ref):
    u = lax.dot_general(u_ref[...], d_ref[0], (((1,), (0,)), ((), ())),
                        preferred_element_type=jnp.float32)
    y = lax.dot_general(u, w_ref[...], (((1,), (0,)), ((), ())),
                        preferred_element_type=jnp.float32)
    o_ref[0] = y


def _tc_c(u0, dd3, wd2):
    return pl.pallas_call(
        _tc_c_body,
        grid=(_MBC, B),
        in_specs=[
            pl.BlockSpec((_TMC, P1), lambda m, b: (m, 0)),
            pl.BlockSpec((1, P1, F_DEC0), lambda m, b: (b, 0, 0)),
            pl.BlockSpec((F_DEC0, S * F_OUT), lambda m, b: (0, 0)),
        ],
        out_specs=pl.BlockSpec((1, _TMC, S * F_OUT), lambda m, b: (b, m, 0)),
        out_shape=jax.ShapeDtypeStruct((B, P0, S * F_OUT), jnp.float32),
    )(u0, dd3, wd2)


def kernel(x, spirals0, W_enc, b_enc, D0, W_fc_enc, b_fc_enc, W_fc_dec,
           b_fc_dec, U0, W_dec, b_dec):
    sp_flat = spirals0.astype(jnp.int32).reshape(-1)

    d0m = D0.at[:, P0 - 1].set(0.0)
    wd2 = W_dec.reshape(F_OUT, S, F_DEC0).transpose(2, 1, 0).reshape(
        F_DEC0, S * F_OUT)

    g0 = _sc_enc_gather(x, sp_flat)
    hd = _tc_a(g0.reshape(B, P0, S * F_IN), W_enc, b_enc.reshape(1, F_ENC),
               d0m)
    z = _tc_b1(hd.reshape(B, P1 * F_ENC), W_fc_enc,
               b_fc_enc.reshape(1, LATENT))
    dd = _tc_b2(z, W_fc_dec, b_fc_dec.reshape(1, P1 * F_DEC0))
    y = _tc_c(U0, dd.reshape(B, P1, F_DEC0), wd2)
    out = _sc_dec_bag(y.reshape(B * P0 * S, F_OUT), sp_flat, b_dec)
    return out


# speedup vs baseline: 10.3096x; 1.0003x over previous
import functools

import jax
import jax.numpy as jnp
from jax import lax
from jax.experimental import pallas as pl
from jax.experimental.pallas import tpu as pltpu, tpu_sc as plsc

B = 8
P0 = 16384
P1 = 1024
S = 16
F_IN = 16
F_ENC = 128
LATENT = 256
F_DEC0 = 128
F_OUT = 16

NW = 32
_STREAM = 128
_PW = P0 // NW
_PCH = 128
_NCH = _PW // _PCH
_GROWS = _PCH * S
_FIRES = _GROWS // _STREAM


def _sc_mesh():
    return plsc.VectorSubcoreMesh(core_axis_name="c", subcore_axis_name="s")


def _worker_id():
    return lax.axis_index("s") * 2 + lax.axis_index("c")


@functools.partial(
    pl.kernel,
    mesh=_sc_mesh(),
    out_type=jax.ShapeDtypeStruct((B * P0 * S, F_IN), jnp.float32),
    compiler_params=pltpu.CompilerParams(use_tc_tiling_on_sc=False),
    scratch_types=[
        pltpu.VMEM((_GROWS,), jnp.int32),
        pltpu.VMEM((_GROWS, F_IN), jnp.float32),
        pltpu.SemaphoreType.DMA,
    ],
)
def _sc_enc_gather(x_hbm, sp_hbm, out_hbm, sp_v, rows_v, sem):
    wid = _worker_id()

    def chunk_body(pc, carry):
        pbase = wid * _PW + pc * _PCH
        pltpu.sync_copy(sp_hbm.at[pl.ds(pbase * S, _GROWS)], sp_v)
        for b in range(B):
            copies = []
            for j in range(_FIRES):
                copies.append(pltpu.async_copy(
                    x_hbm.at[b].at[sp_v.at[pl.ds(j * _STREAM, _STREAM)]],
                    rows_v.at[pl.ds(j * _STREAM, _STREAM)],
                    sem))
            for c in copies:
                c.wait()
            pltpu.sync_copy(
                rows_v, out_hbm.at[pl.ds((b * P0 + pbase) * S, _GROWS)])
        return carry

    lax.fori_loop(0, _NCH, chunk_body, 0)


@functools.partial(
    pl.kernel,
    mesh=_sc_mesh(),
    out_type=jax.ShapeDtypeStruct((B, P0, F_OUT), jnp.float32),
    compiler_params=pltpu.CompilerParams(use_tc_tiling_on_sc=False),
    scratch_types=[
        pltpu.VMEM((_GROWS,), jnp.int32),
        pltpu.VMEM((_GROWS, F_OUT), jnp.float32),
        pltpu.VMEM((_PCH, F_OUT), jnp.float32),
        pltpu.VMEM((F_OUT,), jnp.float32),
        pltpu.SemaphoreType.DMA,
    ],
)
def _sc_dec_bag(y_hbm, sp_hbm, bdec_hbm, out_hbm,
                idx_v, rows_v, out_v, bias_v, sem):
    wid = _worker_id()
    pltpu.sync_copy(bdec_hbm, bias_v)
    iot = lax.iota(jnp.int32, 16)

    def chunk_body(pc, carry):
        pbase = wid * _PW + pc * _PCH
        pltpu.sync_copy(sp_hbm.at[pl.ds(pbase * S, _GROWS)], idx_v)

        def mk_idx(r, rcarry):
            sl = pl.ds(r * S, S)
            idx_v[sl] = idx_v[sl] * S + iot
            return rcarry

        lax.fori_loop(0, _PCH, mk_idx, 0)
        bias = bias_v[...]
        for b in range(B):
            if b > 0:
                def bump(r, rcarry):
                    sl = pl.ds(r * S, S)
                    idx_v[sl] = idx_v[sl] + (P0 * S)
                    return rcarry
                lax.fori_loop(0, _PCH, bump, 0)
            copies = []
            for j in range(_FIRES):
                copies.append(pltpu.async_copy(
                    y_hbm.at[idx_v.at[pl.ds(j * _STREAM, _STREAM)]],
                    rows_v.at[pl.ds(j * _STREAM, _STREAM)],
                    sem))
            for c in copies:
                c.wait()

            def row_body(r, rcarry):
                acc = bias
                for s in range(S):
                    acc = acc + rows_v[r * S + s]
                keep = jnp.where(pbase + r == P0 - 1,
                                 jnp.float32(0.0), jnp.float32(1.0))
                out_v[r] = acc * keep
                return rcarry

            lax.fori_loop(0, _PCH, row_body, 0)
            pltpu.sync_copy(out_v, out_hbm.at[b].at[pl.ds(pbase, _PCH)])
        return carry

    lax.fori_loop(0, _NCH, chunk_body, 0)


_TKA = 2048
_KBA = P0 // _TKA


def _tc_a_body(g_ref, w_ref, b_ref, d_ref, o_ref):
    k = pl.program_id(0)
    b = pl.program_id(1)
    g = g_ref[0].astype(jnp.bfloat16)
    h = lax.dot_general(g, w_ref[...].astype(jnp.bfloat16),
                        (((1,), (1,)), ((), ())),
                        preferred_element_type=jnp.float32)
    h = h + b_ref[...]
    h = jnp.where(h > 0, h, jnp.exp(jnp.minimum(h, 0.0)) - 1.0)
    c = lax.dot_general(d_ref[...].astype(jnp.bfloat16),
                        h.astype(jnp.bfloat16),
                        (((1,), (0,)), ((), ())),
                        preferred_element_type=jnp.float32)

    @pl.when(k == 0)
    def _():
        o_ref[b] = c

    @pl.when(k != 0)
    def _():
        o_ref[b] = o_ref[b] + c


def _tc_a(g0r, w_enc, b_enc2, d0m):
    return pl.pallas_call(
        _tc_a_body,
        grid=(_KBA, B),
        in_specs=[
            pl.BlockSpec((1, _TKA, S * F_IN), lambda k, b: (b, k, 0)),
            pl.BlockSpec((F_ENC, S * F_IN), lambda k, b: (0, 0)),
            pl.BlockSpec((1, F_ENC), lambda k, b: (0, 0)),
            pl.BlockSpec((P1, _TKA), lambda k, b: (0, k)),
        ],
        out_specs=pl.BlockSpec((B, P1, F_ENC), lambda k, b: (0, 0, 0)),
        out_shape=jax.ShapeDtypeStruct((B, P1, F_ENC), jnp.float32),
    )(g0r, w_enc, b_enc2, d0m)


_CKB = 8192
_KBB = (P1 * F_ENC) // _CKB


def _tc_b1_body(h_ref, w_ref, b_ref, o_ref):
    k = pl.program_id(0)
    c = lax.dot_general(h_ref[...], w_ref[...], (((1,), (1,)), ((), ())),
                        preferred_element_type=jnp.float32)

    @pl.when(k == 0)
    def _():
        o_ref[...] = c + b_ref[...]

    @pl.when(k != 0)
    def _():
        o_ref[...] = o_ref[...] + c


def _tc_b1(hd_flat, w_fc_enc, b_fc_enc2):
    return pl.pallas_call(
        _tc_b1_body,
        grid=(_KBB,),
        in_specs=[
            pl.BlockSpec((B, _CKB), lambda k: (0, k)),
            pl.BlockSpec((LATENT, _CKB), lambda k: (0, k)),
            pl.BlockSpec((1, LATENT), lambda k: (0, 0)),
        ],
        out_specs=pl.BlockSpec((B, LATENT), lambda k: (0, 0)),
        out_shape=jax.ShapeDtypeStruct((B, LATENT), jnp.float32),
    )(hd_flat, w_fc_enc, b_fc_enc2)


_CNB = 8192
_NBB = (P1 * F_DEC0) // _CNB


def _tc_b2_body(z_ref, w_ref, b_ref, o_ref):
    c = lax.dot_general(z_ref[...], w_ref[...], (((1,), (1,)), ((), ())),
                        preferred_element_type=jnp.float32)
    o_ref[...] = c + b_ref[...]


def _tc_b2(z, w_fc_dec, b_fc_dec2):
    return pl.pallas_call(
        _tc_b2_body,
        grid=(_NBB,),
        in_specs=[
            pl.BlockSpec((B, LATENT), lambda n: (0, 0)),
            pl.BlockSpec((_CNB, LATENT), lambda n: (n, 0)),
            pl.BlockSpec((1, _CNB), lambda n: (0, n)),
        ],
        out_specs=pl.BlockSpec((B, _CNB), lambda n: (0, n)),
        out_shape=jax.ShapeDtypeStruct((B, P1 * F_DEC0), jnp.float32),
    )(z, w_fc_dec, b_fc_dec2)


_TMC = 2048
_MBC = P0 // _TMC


def _tc_c_body(u_ref, d_ref, w_ref, o_ref):
    u = lax.dot_general(u_ref[...].astype(jnp.bfloat16),
                        d_ref[0].astype(jnp.bfloat16),
                        (((1,), (0,)), ((), ())),
                        preferred_element_type=jnp.float32)
    y = lax.dot_general(u.astype(jnp.bfloat16),
                        w_ref[...].astype(jnp.bfloat16),
                        (((1,), (0,)), ((), ())),
                        preferred_element_type=jnp.float32)
    o_ref[0] = y


def _tc_c(u0, dd3, wd2):
    return pl.pallas_call(
        _tc_c_body,
        grid=(_MBC, B),
        in_specs=[
            pl.BlockSpec((_TMC, P1), lambda m, b: (m, 0)),
            pl.BlockSpec((1, P1, F_DEC0), lambda m, b: (b, 0, 0)),
            pl.BlockSpec((F_DEC0, S * F_OUT), lambda m, b: (0, 0)),
        ],
        out_specs=pl.BlockSpec((1, _TMC, S * F_OUT), lambda m, b: (b, m, 0)),
        out_shape=jax.ShapeDtypeStruct((B, P0, S * F_OUT), jnp.float32),
    )(u0, dd3, wd2)


def kernel(x, spirals0, W_enc, b_enc, D0, W_fc_enc, b_fc_enc, W_fc_dec,
           b_fc_dec, U0, W_dec, b_dec):
    sp_flat = spirals0.astype(jnp.int32).reshape(-1)

    d0m = D0.at[:, P0 - 1].set(0.0)
    wd2 = W_dec.reshape(F_OUT, S, F_DEC0).transpose(2, 1, 0).reshape(
        F_DEC0, S * F_OUT)

    g0 = _sc_enc_gather(x, sp_flat)
    hd = _tc_a(g0.reshape(B, P0, S * F_IN), W_enc, b_enc.reshape(1, F_ENC),
               d0m)
    z = _tc_b1(hd.reshape(B, P1 * F_ENC), W_fc_enc,
               b_fc_enc.reshape(1, LATENT))
    dd = _tc_b2(z, W_fc_dec, b_fc_dec.reshape(1, P1 * F_DEC0))
    y = _tc_c(U0, dd.reshape(B, P1, F_DEC0), wd2)
    out = _sc_dec_bag(y.reshape(B * P0 * S, F_OUT), sp_flat, b_dec)
    return out
